# baseline (speedup 1.0000x reference)
import jax
import jax.numpy as jnp
from jax import lax
from jax.experimental import pallas as pl
from jax.experimental.pallas import tpu as pltpu
import jax.experimental.pallas.tpu_sc as plsc

NC = 2
NS = 16
NW = NC * NS
CH = 128

_f32 = jnp.float32


def _sc_mesh():
    return plsc.VectorSubcoreMesh(
        core_axis_name="c", subcore_axis_name="s", num_cores=NC, num_subcores=NS
    )


def _sc_deg(dst_p, ones8, zeros8, n_pad):
    e_pad = dst_p.shape[0]
    per_w = e_pad // NW
    nch = per_w // CH
    rps = n_pad // NS
    rch = rps // CH

    def body(dst_hbm, ones_hbm, zeros_hbm, out_hbm, dstb, ones_v, z8_v, acc):
        c = lax.axis_index("c")
        s = lax.axis_index("s")
        wid = s * NC + c
        base = wid * per_w
        row0 = s * rps
        pltpu.sync_copy(ones_hbm, ones_v)
        pltpu.sync_copy(zeros_hbm, z8_v)
        for t in range(rch):
            pltpu.sync_copy(z8_v, acc.at[pl.ds(row0 + t * CH, CH)])
        plsc.subcore_barrier()

        def step(j, carry):
            off = base + j * CH
            pltpu.sync_copy(dst_hbm.at[pl.ds(off, CH)], dstb.at[0])
            pltpu.sync_copy(ones_v, acc.at[dstb.at[0]], add=True)
            return carry

        lax.fori_loop(0, nch, step, 0)
        plsc.subcore_barrier()
        for t in range(rch):
            sl = pl.ds(row0 + t * CH, CH)
            pltpu.sync_copy(acc.at[sl], z8_v)
            pltpu.sync_copy(z8_v, out_hbm.at[c, sl])

    f = pl.kernel(
        body,
        out_type=jax.ShapeDtypeStruct((NC, n_pad, 8), _f32),
        mesh=_sc_mesh(),
        scratch_types=[
            pltpu.VMEM((1, CH), jnp.int32),
            pltpu.VMEM((CH, 8), _f32),
            pltpu.VMEM((CH, 8), _f32),
            pltpu.VMEM_SHARED((n_pad, 8), _f32),
        ],
        compiler_params=pltpu.CompilerParams(use_tc_tiling_on_sc=False),
    )
    return f(dst_p, ones8, zeros8)


def _sc_agg(table, idxc, zrows, n_pad):
    n, d = table.shape
    nch = idxc.shape[0] // NW
    assert nch % 4 == 0 and nch >= 8
    rps = n_pad // NS
    rch = rps // CH

    def body(table_hbm, idx_hbm, zeros_hbm, out_hbm,
             idx0, idx1, idx2, idx3, rows0, rows1,
             is0, is1, is2, is3, gs0, gs1, ss0, ss1, acc):
        c = lax.axis_index("c")
        s = lax.axis_index("s")
        wid = s * NC + c
        base = wid * nch
        row0 = s * rps
        pltpu.sync_copy(zeros_hbm, rows0)
        for t in range(rch):
            pltpu.sync_copy(rows0, acc.at[pl.ds(row0 + t * CH, CH)])
        plsc.subcore_barrier()

        rows = (rows0, rows1)
        idxb = (idx0, idx1, idx2, idx3)
        isem = (is0, is1, is2, is3)
        gsem = (gs0, gs1)
        ssem = (ss0, ss1)

        def fetch_idx(j, k):
            pltpu.async_copy(idx_hbm.at[base + j], idxb[k % 4], isem[k % 4])

        def wait_idx(j, k):
            pltpu.make_async_copy(idx_hbm.at[base + j], idxb[k % 4],
                                  isem[k % 4]).wait()

        def start_gather(k):
            pltpu.async_copy(table_hbm.at[idxb[k % 4].at[0]], rows[k % 2],
                             gsem[k % 2])

        def wait_gather(k):
            pltpu.make_async_copy(table_hbm.at[idxb[k % 4].at[0]], rows[k % 2],
                                  gsem[k % 2]).wait()

        def start_scatter(k):
            pltpu.async_copy(rows[k % 2], acc.at[idxb[k % 4].at[1]],
                             ssem[k % 2], add=True)

        def drain_scatter(k):
            pltpu.make_async_copy(rows[k % 2], acc.at[idxb[k % 4].at[1]],
                                  ssem[k % 2]).wait()

        def steady(j, k, do_fetch=True, do_next_gather=True, do_drain=True):
            wait_gather(k)
            start_scatter(k)
            if do_next_gather:
                wait_idx(j + 1, k + 1)
                if do_drain:
                    drain_scatter(k + 1)
                start_gather(k + 1)
            if do_fetch:
                fetch_idx(j + 3, k + 3)

        fetch_idx(0, 0)
        fetch_idx(1, 1)
        fetch_idx(2, 2)
        wait_idx(0, 0)
        start_gather(0)
        steady(0, 0, do_drain=False)

        def step(g, carry):
            j = 1 + g * 4
            steady(j + 0, 1)
            steady(j + 1, 2)
            steady(j + 2, 3)
            steady(j + 3, 0)
            return carry

        lax.fori_loop(0, (nch - 4) // 4, step, 0)
        steady(nch - 3, nch - 3, do_fetch=False)
        steady(nch - 2, nch - 2, do_fetch=False)
        steady(nch - 1, nch - 1, do_fetch=False, do_next_gather=False)
        drain_scatter(nch - 2)
        drain_scatter(nch - 1)

        plsc.subcore_barrier()
        for t in range(rch):
            sl = pl.ds(row0 + t * CH, CH)
            pltpu.sync_copy(acc.at[sl], rows0)
            pltpu.sync_copy(rows0, out_hbm.at[c, sl])

    f = pl.kernel(
        body,
        out_type=jax.ShapeDtypeStruct((NC, n_pad, d), _f32),
        mesh=_sc_mesh(),
        scratch_types=(
            [pltpu.VMEM((2, CH), jnp.int32)] * 4
            + [pltpu.VMEM((CH, d), _f32)] * 2
            + [pltpu.SemaphoreType.DMA] * 8
            + [pltpu.VMEM_SHARED((n_pad, d), _f32)]
        ),
        compiler_params=pltpu.CompilerParams(use_tc_tiling_on_sc=False),
    )
    return f(table, idxc, zrows)


def _block(n):
    for b in (1000, 500, 250, 200, 125, 100, 50, 25, 10, 5):
        if n % b == 0:
            return b
    return n


def _tc_l1(x, w1t, d0, d1):
    n, dx = x.shape
    dd = w1t.shape[1]
    bs = _block(n)

    def body(x_ref, w_ref, d0_ref, d1_ref, hp_ref, dinv_ref):
        deg = d0_ref[:, 0:1] + d1_ref[:, 0:1] + 1.0
        dinv = lax.rsqrt(deg)
        hp_ref[...] = (
            jnp.dot(x_ref[...], w_ref[...], preferred_element_type=_f32) * dinv
        )
        dinv_ref[...] = jnp.broadcast_to(dinv, dinv_ref.shape)

    return pl.pallas_call(
        body,
        grid=(n // bs,),
        in_specs=[
            pl.BlockSpec((bs, dx), lambda i: (i, 0)),
            pl.BlockSpec((dx, dd), lambda i: (0, 0)),
            pl.BlockSpec((bs, 8), lambda i: (i, 0)),
            pl.BlockSpec((bs, 8), lambda i: (i, 0)),
        ],
        out_specs=[
            pl.BlockSpec((bs, dd), lambda i: (i, 0)),
            pl.BlockSpec((bs, 8), lambda i: (i, 0)),
        ],
        out_shape=[
            jax.ShapeDtypeStruct((n, dd), _f32),
            jax.ShapeDtypeStruct((n, 8), _f32),
        ],
    )(x, w1t, d0, d1)


def _tc_combine(p0, p1, hp, dinv8, b):
    n, d = hp.shape
    bs = _block(n)

    def body(p0_ref, p1_ref, hp_ref, dinv_ref, b_ref, z_ref, st_ref):
        i = pl.program_id(0)
        z = dinv_ref[:, 0:1] * (p0_ref[...] + p1_ref[...] + hp_ref[...]) + b_ref[...]
        z_ref[...] = z
        part = jnp.concatenate(
            [jnp.sum(z, 0, keepdims=True), jnp.sum(z * z, 0, keepdims=True)], 0
        )

        @pl.when(i == 0)
        def _init():
            st_ref[...] = part

        @pl.when(i > 0)
        def _acc():
            st_ref[...] += part

    return pl.pallas_call(
        body,
        grid=(n // bs,),
        in_specs=[
            pl.BlockSpec((bs, d), lambda i: (i, 0)),
            pl.BlockSpec((bs, d), lambda i: (i, 0)),
            pl.BlockSpec((bs, d), lambda i: (i, 0)),
            pl.BlockSpec((bs, 8), lambda i: (i, 0)),
            pl.BlockSpec((1, d), lambda i: (0, 0)),
        ],
        out_specs=[
            pl.BlockSpec((bs, d), lambda i: (i, 0)),
            pl.BlockSpec((2, d), lambda i: (0, 0)),
        ],
        out_shape=[
            jax.ShapeDtypeStruct((n, d), _f32),
            jax.ShapeDtypeStruct((2, d), _f32),
        ],
    )(p0, p1, hp, dinv8, b)


def _tc_bnmm(z, st, g, beta, wt, dinv8=None, bias=None):
    n, d = z.shape
    do = wt.shape[1]
    bs = _block(n)
    nf = float(n)

    def body(*refs):
        if dinv8 is not None:
            z_ref, st_ref, g_ref, beta_ref, w_ref, dv_ref, o_ref = refs
        elif bias is not None:
            z_ref, st_ref, g_ref, beta_ref, w_ref, b_ref, o_ref = refs
        else:
            z_ref, st_ref, g_ref, beta_ref, w_ref, o_ref = refs
        mean = st_ref[0:1, :] / nf
        var = st_ref[1:2, :] / nf - mean * mean
        sc = g_ref[...] * lax.rsqrt(var + 1e-5)
        h = jnp.maximum(z_ref[...] * sc + (beta_ref[...] - mean * sc), 0.0)
        o = jnp.dot(h, w_ref[...], preferred_element_type=_f32)
        if dinv8 is not None:
            o = o * dv_ref[:, 0:1]
        if bias is not None:
            o = o + b_ref[...]
        o_ref[...] = o

    in_specs = [
        pl.BlockSpec((bs, d), lambda i: (i, 0)),
        pl.BlockSpec((2, d), lambda i: (0, 0)),
        pl.BlockSpec((1, d), lambda i: (0, 0)),
        pl.BlockSpec((1, d), lambda i: (0, 0)),
        pl.BlockSpec((d, do), lambda i: (0, 0)),
    ]
    args = [z, st, g, beta, wt]
    if dinv8 is not None:
        in_specs.append(pl.BlockSpec((bs, 8), lambda i: (i, 0)))
        args.append(dinv8)
    if bias is not None:
        in_specs.append(pl.BlockSpec((1, do), lambda i: (0, 0)))
        args.append(bias)

    return pl.pallas_call(
        body,
        grid=(n // bs,),
        in_specs=in_specs,
        out_specs=pl.BlockSpec((bs, do), lambda i: (i, 0)),
        out_shape=jax.ShapeDtypeStruct((n, do), _f32),
    )(*args)


def _ceil_to(a, m):
    return (a + m - 1) // m * m


def kernel(x, edge_index, W1, b1, g1, beta1, W2, b2, g2, beta2, Wfc, bfc):
    n = x.shape[0]
    e = edge_index.shape[1]
    n_pad = _ceil_to(n + 1, NS * CH)
    e_pad = _ceil_to(e, NW * CH * 8)

    src_p = jnp.concatenate(
        [edge_index[0], jnp.zeros((e_pad - e,), edge_index.dtype)]
    )
    dst_p = jnp.concatenate(
        [edge_index[1], jnp.full((e_pad - e,), n, edge_index.dtype)]
    )

    idxc = jnp.stack([src_p.reshape(-1, CH), dst_p.reshape(-1, CH)], axis=1)

    ones8 = jnp.ones((CH, 8), _f32)
    zeros8 = jnp.zeros((CH, 8), _f32)
    degp = _sc_deg(dst_p, ones8, zeros8, n_pad)

    hp1, dinv8 = _tc_l1(x, W1.T, degp[0, :n], degp[1, :n])

    parts1 = _sc_agg(hp1, idxc, jnp.zeros((CH, hp1.shape[1]), _f32), n_pad)
    z1, st1 = _tc_combine(parts1[0, :n], parts1[1, :n], hp1, dinv8, b1.reshape(1, -1))
    hp2 = _tc_bnmm(z1, st1, g1.reshape(1, -1), beta1.reshape(1, -1), W2.T, dinv8=dinv8)

    parts2 = _sc_agg(hp2, idxc, jnp.zeros((CH, hp2.shape[1]), _f32), n_pad)
    z2, st2 = _tc_combine(parts2[0, :n], parts2[1, :n], hp2, dinv8, b2.reshape(1, -1))
    return _tc_bnmm(z2, st2, g2.reshape(1, -1), beta2.reshape(1, -1), Wfc.T, bias=bfc.reshape(1, -1))


# speedup vs baseline: 2.0423x; 2.0423x over previous
import jax
import jax.numpy as jnp
from jax import lax
from jax.experimental import pallas as pl
from jax.experimental.pallas import tpu as pltpu
import jax.experimental.pallas.tpu_sc as plsc

NC = 2
NS = 16
NW = NC * NS
CH = 128

_f32 = jnp.float32


def _sc_mesh():
    return plsc.VectorSubcoreMesh(
        core_axis_name="c", subcore_axis_name="s", num_cores=NC, num_subcores=NS
    )


def _sc_deg(dst_p, ones8, zeros8, n_pad):
    e_pad = dst_p.shape[0]
    per_w = e_pad // NW
    nch = per_w // CH
    rps = n_pad // NS
    rch = rps // CH

    def body(dst_hbm, ones_hbm, zeros_hbm, out_hbm, dstb, ones_v, z8_v, acc):
        c = lax.axis_index("c")
        s = lax.axis_index("s")
        wid = s * NC + c
        base = wid * per_w
        row0 = s * rps
        pltpu.sync_copy(ones_hbm, ones_v)
        pltpu.sync_copy(zeros_hbm, z8_v)
        for t in range(rch):
            pltpu.sync_copy(z8_v, acc.at[pl.ds(row0 + t * CH, CH)])
        plsc.subcore_barrier()

        def step(j, carry):
            off = base + j * CH
            pltpu.sync_copy(dst_hbm.at[pl.ds(off, CH)], dstb.at[0])
            pltpu.sync_copy(ones_v, acc.at[dstb.at[0]], add=True)
            return carry

        lax.fori_loop(0, nch, step, 0)
        plsc.subcore_barrier()
        for t in range(rch):
            sl = pl.ds(row0 + t * CH, CH)
            pltpu.sync_copy(acc.at[sl], z8_v)
            pltpu.sync_copy(z8_v, out_hbm.at[c, sl])

    f = pl.kernel(
        body,
        out_type=jax.ShapeDtypeStruct((NC, n_pad, 8), _f32),
        mesh=_sc_mesh(),
        scratch_types=[
            pltpu.VMEM((1, CH), jnp.int32),
            pltpu.VMEM((CH, 8), _f32),
            pltpu.VMEM((CH, 8), _f32),
            pltpu.VMEM_SHARED((n_pad, 8), _f32),
        ],
        compiler_params=pltpu.CompilerParams(use_tc_tiling_on_sc=False),
    )
    return f(dst_p, ones8, zeros8)


SPLIT0 = 0.68


def _sc_agg(table, idxc, zrows, n_pad):
    n, d = table.shape
    pairs = idxc.shape[0] // NS
    nch0 = max(2, int(pairs * SPLIT0) // 2 * 2)
    nch1 = pairs - nch0
    assert nch1 >= 2
    rps = n_pad // NS
    rch = rps // CH

    def body(table_hbm, idx_hbm, zeros_hbm, out_hbm,
             idx0, idx1, rows0, rows1, sem0, sem1, acc):
        c = lax.axis_index("c")
        s = lax.axis_index("s")
        row0 = s * rps
        pltpu.sync_copy(zeros_hbm, rows0)
        for t in range(rch):
            pltpu.sync_copy(rows0, acc.at[pl.ds(row0 + t * CH, CH)])
        plsc.subcore_barrier()

        rows = (rows0, rows1)
        idxb = (idx0, idx1)
        sems = (sem0, sem1)

        def make_pipeline(nch, base):
            def fetch_start(j, slot):
                pltpu.sync_copy(idx_hbm.at[base + j], idxb[slot])
                pltpu.async_copy(table_hbm.at[idxb[slot].at[0]], rows[slot],
                                 sems[slot])

            def finish(slot):
                pltpu.make_async_copy(table_hbm.at[idxb[slot].at[0]],
                                      rows[slot], sems[slot]).wait()
                pltpu.sync_copy(rows[slot], acc.at[idxb[slot].at[1]], add=True)

            def run():
                fetch_start(0, 0)

                def step(p, carry):
                    j = p * 2
                    fetch_start(j + 1, 1)
                    finish(0)
                    fetch_start(j + 2, 0)
                    finish(1)
                    return carry

                lax.fori_loop(0, nch // 2 - 1, step, 0)
                fetch_start(nch - 1, 1)
                finish(0)
                finish(1)

            return run

        run0 = make_pipeline(nch0, s * nch0)
        run1 = make_pipeline(nch1, NS * nch0 + s * nch1)
        lax.cond(c == 0, run0, run1)

        plsc.subcore_barrier()
        for t in range(rch):
            sl = pl.ds(row0 + t * CH, CH)
            pltpu.sync_copy(acc.at[sl], rows0)
            pltpu.sync_copy(rows0, out_hbm.at[c, sl])

    f = pl.kernel(
        body,
        out_type=jax.ShapeDtypeStruct((NC, n_pad, d), _f32),
        mesh=_sc_mesh(),
        scratch_types=[
            pltpu.VMEM((2, CH), jnp.int32),
            pltpu.VMEM((2, CH), jnp.int32),
            pltpu.VMEM((CH, d), _f32),
            pltpu.VMEM((CH, d), _f32),
            pltpu.SemaphoreType.DMA,
            pltpu.SemaphoreType.DMA,
            pltpu.VMEM_SHARED((n_pad, d), _f32),
        ],
        compiler_params=pltpu.CompilerParams(use_tc_tiling_on_sc=False),
    )
    return f(table, idxc, zrows)


def _block(n):
    for b in (1000, 500, 250, 200, 125, 100, 50, 25, 10, 5):
        if n % b == 0:
            return b
    return n


def _tc_l1(x, w1t, degp):
    n, dx = x.shape
    dd = w1t.shape[1]
    bs = _block(n)

    def body(x_ref, w_ref, d0_ref, d1_ref, hp_ref, dinv_ref):
        deg = d0_ref[0, :, 0:1] + d1_ref[0, :, 0:1] + 1.0
        dinv = lax.rsqrt(deg)
        hp_ref[...] = (
            jnp.dot(x_ref[...], w_ref[...], preferred_element_type=_f32) * dinv
        )
        dinv_ref[...] = jnp.broadcast_to(dinv, dinv_ref.shape)

    return pl.pallas_call(
        body,
        grid=(n // bs,),
        in_specs=[
            pl.BlockSpec((bs, dx), lambda i: (i, 0)),
            pl.BlockSpec((dx, dd), lambda i: (0, 0)),
            pl.BlockSpec((1, bs, 8), lambda i: (0, i, 0)),
            pl.BlockSpec((1, bs, 8), lambda i: (1, i, 0)),
        ],
        out_specs=[
            pl.BlockSpec((bs, dd), lambda i: (i, 0)),
            pl.BlockSpec((bs, 8), lambda i: (i, 0)),
        ],
        out_shape=[
            jax.ShapeDtypeStruct((n, dd), _f32),
            jax.ShapeDtypeStruct((n, 8), _f32),
        ],
    )(x, w1t, degp, degp)


def _tc_combine(parts, hp, dinv8, b):
    n, d = hp.shape
    bs = _block(n)

    def body(p0_ref, p1_ref, hp_ref, dinv_ref, b_ref, z_ref, st_ref):
        i = pl.program_id(0)
        z = (dinv_ref[:, 0:1] * (p0_ref[0] + p1_ref[0] + hp_ref[...])
             + b_ref[...])
        z_ref[...] = z
        part = jnp.concatenate(
            [jnp.sum(z, 0, keepdims=True), jnp.sum(z * z, 0, keepdims=True)], 0
        )

        @pl.when(i == 0)
        def _init():
            st_ref[...] = part

        @pl.when(i > 0)
        def _acc():
            st_ref[...] += part

    return pl.pallas_call(
        body,
        grid=(n // bs,),
        in_specs=[
            pl.BlockSpec((1, bs, d), lambda i: (0, i, 0)),
            pl.BlockSpec((1, bs, d), lambda i: (1, i, 0)),
            pl.BlockSpec((bs, d), lambda i: (i, 0)),
            pl.BlockSpec((bs, 8), lambda i: (i, 0)),
            pl.BlockSpec((1, d), lambda i: (0, 0)),
        ],
        out_specs=[
            pl.BlockSpec((bs, d), lambda i: (i, 0)),
            pl.BlockSpec((2, d), lambda i: (0, 0)),
        ],
        out_shape=[
            jax.ShapeDtypeStruct((n, d), _f32),
            jax.ShapeDtypeStruct((2, d), _f32),
        ],
    )(parts, parts, hp, dinv8, b)


def _tc_bnmm(z, st, g, beta, wt, dinv8=None, bias=None):
    n, d = z.shape
    do = wt.shape[1]
    bs = _block(n)
    nf = float(n)

    def body(*refs):
        if dinv8 is not None:
            z_ref, st_ref, g_ref, beta_ref, w_ref, dv_ref, o_ref = refs
        elif bias is not None:
            z_ref, st_ref, g_ref, beta_ref, w_ref, b_ref, o_ref = refs
        else:
            z_ref, st_ref, g_ref, beta_ref, w_ref, o_ref = refs
        mean = st_ref[0:1, :] / nf
        var = st_ref[1:2, :] / nf - mean * mean
        sc = g_ref[...] * lax.rsqrt(var + 1e-5)
        h = jnp.maximum(z_ref[...] * sc + (beta_ref[...] - mean * sc), 0.0)
        o = jnp.dot(h, w_ref[...], preferred_element_type=_f32)
        if dinv8 is not None:
            o = o * dv_ref[:, 0:1]
        if bias is not None:
            o = o + b_ref[...]
        o_ref[...] = o

    in_specs = [
        pl.BlockSpec((bs, d), lambda i: (i, 0)),
        pl.BlockSpec((2, d), lambda i: (0, 0)),
        pl.BlockSpec((1, d), lambda i: (0, 0)),
        pl.BlockSpec((1, d), lambda i: (0, 0)),
        pl.BlockSpec((d, do), lambda i: (0, 0)),
    ]
    args = [z, st, g, beta, wt]
    if dinv8 is not None:
        in_specs.append(pl.BlockSpec((bs, 8), lambda i: (i, 0)))
        args.append(dinv8)
    if bias is not None:
        in_specs.append(pl.BlockSpec((1, do), lambda i: (0, 0)))
        args.append(bias)

    return pl.pallas_call(
        body,
        grid=(n // bs,),
        in_specs=in_specs,
        out_specs=pl.BlockSpec((bs, do), lambda i: (i, 0)),
        out_shape=jax.ShapeDtypeStruct((n, do), _f32),
    )(*args)


def _ceil_to(a, m):
    return (a + m - 1) // m * m


def kernel(x, edge_index, W1, b1, g1, beta1, W2, b2, g2, beta2, Wfc, bfc):
    n = x.shape[0]
    e = edge_index.shape[1]
    n_pad = _ceil_to(n + 1, NS * CH)
    e_pad = _ceil_to(e, NW * CH * 2)

    src_p = jnp.concatenate(
        [edge_index[0], jnp.zeros((e_pad - e,), edge_index.dtype)]
    )
    dst_p = jnp.concatenate(
        [edge_index[1], jnp.full((e_pad - e,), n, edge_index.dtype)]
    )

    idxc = jnp.stack([src_p.reshape(-1, CH), dst_p.reshape(-1, CH)], axis=1)

    ones8 = jnp.ones((CH, 8), _f32)
    zeros8 = jnp.zeros((CH, 8), _f32)
    degp = _sc_deg(dst_p, ones8, zeros8, n_pad)

    hp1, dinv8 = _tc_l1(x, W1.T, degp)

    parts1 = _sc_agg(hp1, idxc, jnp.zeros((CH, hp1.shape[1]), _f32), n_pad)
    z1, st1 = _tc_combine(parts1, hp1, dinv8, b1.reshape(1, -1))
    hp2 = _tc_bnmm(z1, st1, g1.reshape(1, -1), beta1.reshape(1, -1), W2.T, dinv8=dinv8)

    parts2 = _sc_agg(hp2, idxc, jnp.zeros((CH, hp2.shape[1]), _f32), n_pad)
    z2, st2 = _tc_combine(parts2, hp2, dinv8, b2.reshape(1, -1))
    return _tc_bnmm(z2, st2, g2.reshape(1, -1), beta2.reshape(1, -1), Wfc.T, bias=bfc.reshape(1, -1))


# speedup vs baseline: 2.0877x; 1.0222x over previous
import jax
import jax.numpy as jnp
from jax import lax
from jax.experimental import pallas as pl
from jax.experimental.pallas import tpu as pltpu
import jax.experimental.pallas.tpu_sc as plsc

NC = 2
NS = 16
NW = NC * NS
CH = 128

_f32 = jnp.float32


def _sc_mesh():
    return plsc.VectorSubcoreMesh(
        core_axis_name="c", subcore_axis_name="s", num_cores=NC, num_subcores=NS
    )


def _sc_deg(idxc, ones8, zeros8, n_pad):
    nch = idxc.shape[0] // NW
    assert nch % 2 == 0
    rps = n_pad // NS
    rch = rps // CH

    def body(idx_hbm, ones_hbm, zeros_hbm, out_hbm,
             idx0, idx1, ones_v, z8_v, is0, is1, acc):
        c = lax.axis_index("c")
        s = lax.axis_index("s")
        wid = s * NC + c
        base = wid * nch
        row0 = s * rps
        pltpu.sync_copy(ones_hbm, ones_v)
        pltpu.sync_copy(zeros_hbm, z8_v)
        for t in range(rch):
            pltpu.sync_copy(z8_v, acc.at[pl.ds(row0 + t * CH, CH)])
        plsc.subcore_barrier()

        idxb = (idx0, idx1)
        isem = (is0, is1)

        def fetch(j, slot):
            pltpu.async_copy(idx_hbm.at[base + j], idxb[slot], isem[slot])

        def wait_fetch(j, slot):
            pltpu.make_async_copy(idx_hbm.at[base + j], idxb[slot],
                                  isem[slot]).wait()

        def scat(slot):
            pltpu.sync_copy(ones_v, acc.at[idxb[slot].at[1]], add=True)

        fetch(0, 0)
        fetch(1, 1)

        def step(p, carry):
            j = p * 2
            wait_fetch(j, 0)
            scat(0)
            fetch(j + 2, 0)
            wait_fetch(j + 1, 1)
            scat(1)
            fetch(j + 3, 1)
            return carry

        lax.fori_loop(0, nch // 2 - 1, step, 0)
        wait_fetch(nch - 2, 0)
        scat(0)
        wait_fetch(nch - 1, 1)
        scat(1)

        plsc.subcore_barrier()
        for t in range(rch):
            sl = pl.ds(row0 + t * CH, CH)
            pltpu.sync_copy(acc.at[sl], z8_v)
            pltpu.sync_copy(z8_v, out_hbm.at[c, sl])

    f = pl.kernel(
        body,
        out_type=jax.ShapeDtypeStruct((NC, n_pad, 8), _f32),
        mesh=_sc_mesh(),
        scratch_types=[
            pltpu.VMEM((2, CH), jnp.int32),
            pltpu.VMEM((2, CH), jnp.int32),
            pltpu.VMEM((CH, 8), _f32),
            pltpu.VMEM((CH, 8), _f32),
            pltpu.SemaphoreType.DMA,
            pltpu.SemaphoreType.DMA,
            pltpu.VMEM_SHARED((n_pad, 8), _f32),
        ],
        compiler_params=pltpu.CompilerParams(use_tc_tiling_on_sc=False),
    )
    return f(idxc, ones8, zeros8)


def _sc_agg(table, idxc, zrows, n_pad, split0):
    n, d = table.shape
    pairs = idxc.shape[0] // NS
    nch0 = max(2, int(pairs * split0) // 2 * 2)
    nch1 = pairs - nch0
    assert nch1 >= 2
    rps = n_pad // NS
    rch = rps // CH

    def body(table_hbm, idx_hbm, zeros_hbm, out_hbm,
             idx0, idx1, rows0, rows1, sem0, sem1, acc):
        c = lax.axis_index("c")
        s = lax.axis_index("s")
        row0 = s * rps
        pltpu.sync_copy(zeros_hbm, rows0)
        for t in range(rch):
            pltpu.sync_copy(rows0, acc.at[pl.ds(row0 + t * CH, CH)])
        plsc.subcore_barrier()

        rows = (rows0, rows1)
        idxb = (idx0, idx1)
        sems = (sem0, sem1)

        def make_pipeline(nch, base):
            def fetch_start(j, slot):
                pltpu.sync_copy(idx_hbm.at[base + j], idxb[slot])
                pltpu.async_copy(table_hbm.at[idxb[slot].at[0]], rows[slot],
                                 sems[slot])

            def finish(slot):
                pltpu.make_async_copy(table_hbm.at[idxb[slot].at[0]],
                                      rows[slot], sems[slot]).wait()
                pltpu.sync_copy(rows[slot], acc.at[idxb[slot].at[1]], add=True)

            def run():
                fetch_start(0, 0)

                def step(p, carry):
                    j = p * 2
                    fetch_start(j + 1, 1)
                    finish(0)
                    fetch_start(j + 2, 0)
                    finish(1)
                    return carry

                lax.fori_loop(0, nch // 2 - 1, step, 0)
                fetch_start(nch - 1, 1)
                finish(0)
                finish(1)

            return run

        run0 = make_pipeline(nch0, s * nch0)
        run1 = make_pipeline(nch1, NS * nch0 + s * nch1)
        lax.cond(c == 0, run0, run1)

        plsc.subcore_barrier()
        for t in range(rch):
            sl = pl.ds(row0 + t * CH, CH)
            pltpu.sync_copy(acc.at[sl], rows0)
            pltpu.sync_copy(rows0, out_hbm.at[c, sl])

    f = pl.kernel(
        body,
        out_type=jax.ShapeDtypeStruct((NC, n_pad, d), _f32),
        mesh=_sc_mesh(),
        scratch_types=[
            pltpu.VMEM((2, CH), jnp.int32),
            pltpu.VMEM((2, CH), jnp.int32),
            pltpu.VMEM((CH, d), _f32),
            pltpu.VMEM((CH, d), _f32),
            pltpu.SemaphoreType.DMA,
            pltpu.SemaphoreType.DMA,
            pltpu.VMEM_SHARED((n_pad, d), _f32),
        ],
        compiler_params=pltpu.CompilerParams(use_tc_tiling_on_sc=False),
    )
    return f(table, idxc, zrows)


def _block(n):
    for b in (1000, 500, 250, 200, 125, 100, 50, 25, 10, 5):
        if n % b == 0:
            return b
    return n


def _tc_l1(x, w1t, degp):
    n, dx = x.shape
    dd = w1t.shape[1]
    bs = _block(n)

    def body(x_ref, w_ref, d0_ref, d1_ref, hp_ref, dinv_ref):
        deg = d0_ref[0, :, 0:1] + d1_ref[0, :, 0:1] + 1.0
        dinv = lax.rsqrt(deg)
        hp_ref[...] = (
            jnp.dot(x_ref[...], w_ref[...], preferred_element_type=_f32) * dinv
        )
        dinv_ref[...] = jnp.broadcast_to(dinv, dinv_ref.shape)

    return pl.pallas_call(
        body,
        grid=(n // bs,),
        in_specs=[
            pl.BlockSpec((bs, dx), lambda i: (i, 0)),
            pl.BlockSpec((dx, dd), lambda i: (0, 0)),
            pl.BlockSpec((1, bs, 8), lambda i: (0, i, 0)),
            pl.BlockSpec((1, bs, 8), lambda i: (1, i, 0)),
        ],
        out_specs=[
            pl.BlockSpec((bs, dd), lambda i: (i, 0)),
            pl.BlockSpec((bs, 8), lambda i: (i, 0)),
        ],
        out_shape=[
            jax.ShapeDtypeStruct((n, dd), _f32),
            jax.ShapeDtypeStruct((n, 8), _f32),
        ],
    )(x, w1t, degp, degp)


def _tc_combine(parts, hp, dinv8, b):
    n, d = hp.shape
    bs = _block(n)

    def body(p0_ref, p1_ref, hp_ref, dinv_ref, b_ref, z_ref, st_ref):
        i = pl.program_id(0)
        z = (dinv_ref[:, 0:1] * (p0_ref[0] + p1_ref[0] + hp_ref[...])
             + b_ref[...])
        z_ref[...] = z
        part = jnp.concatenate(
            [jnp.sum(z, 0, keepdims=True), jnp.sum(z * z, 0, keepdims=True)], 0
        )

        @pl.when(i == 0)
        def _init():
            st_ref[...] = part

        @pl.when(i > 0)
        def _acc():
            st_ref[...] += part

    return pl.pallas_call(
        body,
        grid=(n // bs,),
        in_specs=[
            pl.BlockSpec((1, bs, d), lambda i: (0, i, 0)),
            pl.BlockSpec((1, bs, d), lambda i: (1, i, 0)),
            pl.BlockSpec((bs, d), lambda i: (i, 0)),
            pl.BlockSpec((bs, 8), lambda i: (i, 0)),
            pl.BlockSpec((1, d), lambda i: (0, 0)),
        ],
        out_specs=[
            pl.BlockSpec((bs, d), lambda i: (i, 0)),
            pl.BlockSpec((2, d), lambda i: (0, 0)),
        ],
        out_shape=[
            jax.ShapeDtypeStruct((n, d), _f32),
            jax.ShapeDtypeStruct((2, d), _f32),
        ],
    )(parts, parts, hp, dinv8, b)


def _tc_bnmm(z, st, g, beta, wt, dinv8=None, bias=None):
    n, d = z.shape
    do = wt.shape[1]
    bs = _block(n)
    nf = float(n)

    def body(*refs):
        if dinv8 is not None:
            z_ref, st_ref, g_ref, beta_ref, w_ref, dv_ref, o_ref = refs
        elif bias is not None:
            z_ref, st_ref, g_ref, beta_ref, w_ref, b_ref, o_ref = refs
        else:
            z_ref, st_ref, g_ref, beta_ref, w_ref, o_ref = refs
        mean = st_ref[0:1, :] / nf
        var = st_ref[1:2, :] / nf - mean * mean
        sc = g_ref[...] * lax.rsqrt(var + 1e-5)
        h = jnp.maximum(z_ref[...] * sc + (beta_ref[...] - mean * sc), 0.0)
        o = jnp.dot(h, w_ref[...], preferred_element_type=_f32)
        if dinv8 is not None:
            o = o * dv_ref[:, 0:1]
        if bias is not None:
            o = o + b_ref[...]
        o_ref[...] = o

    in_specs = [
        pl.BlockSpec((bs, d), lambda i: (i, 0)),
        pl.BlockSpec((2, d), lambda i: (0, 0)),
        pl.BlockSpec((1, d), lambda i: (0, 0)),
        pl.BlockSpec((1, d), lambda i: (0, 0)),
        pl.BlockSpec((d, do), lambda i: (0, 0)),
    ]
    args = [z, st, g, beta, wt]
    if dinv8 is not None:
        in_specs.append(pl.BlockSpec((bs, 8), lambda i: (i, 0)))
        args.append(dinv8)
    if bias is not None:
        in_specs.append(pl.BlockSpec((1, do), lambda i: (0, 0)))
        args.append(bias)

    return pl.pallas_call(
        body,
        grid=(n // bs,),
        in_specs=in_specs,
        out_specs=pl.BlockSpec((bs, do), lambda i: (i, 0)),
        out_shape=jax.ShapeDtypeStruct((n, do), _f32),
    )(*args)


def _ceil_to(a, m):
    return (a + m - 1) // m * m


def kernel(x, edge_index, W1, b1, g1, beta1, W2, b2, g2, beta2, Wfc, bfc):
    n = x.shape[0]
    e = edge_index.shape[1]
    n_pad = _ceil_to(n + 1, NS * CH)
    e_pad = _ceil_to(e, NW * CH * 2)

    src_p = jnp.concatenate(
        [edge_index[0], jnp.zeros((e_pad - e,), edge_index.dtype)]
    )
    dst_p = jnp.concatenate(
        [edge_index[1], jnp.full((e_pad - e,), n, edge_index.dtype)]
    )

    idxc = jnp.stack([src_p.reshape(-1, CH), dst_p.reshape(-1, CH)], axis=1)

    ones8 = jnp.ones((CH, 8), _f32)
    zeros8 = jnp.zeros((CH, 8), _f32)
    degp = _sc_deg(idxc, ones8, zeros8, n_pad)

    hp1, dinv8 = _tc_l1(x, W1.T, degp)

    parts1 = _sc_agg(hp1, idxc, jnp.zeros((CH, hp1.shape[1]), _f32), n_pad, 0.74)
    z1, st1 = _tc_combine(parts1, hp1, dinv8, b1.reshape(1, -1))
    hp2 = _tc_bnmm(z1, st1, g1.reshape(1, -1), beta1.reshape(1, -1), W2.T, dinv8=dinv8)

    parts2 = _sc_agg(hp2, idxc, jnp.zeros((CH, hp2.shape[1]), _f32), n_pad, 0.67)
    z2, st2 = _tc_combine(parts2, hp2, dinv8, b2.reshape(1, -1))
    return _tc_bnmm(z2, st2, g2.reshape(1, -1), beta2.reshape(1, -1), Wfc.T, bias=bfc.reshape(1, -1))


# speedup vs baseline: 2.1544x; 1.0319x over previous
import jax
import jax.numpy as jnp
from jax import lax
from jax.experimental import pallas as pl
from jax.experimental.pallas import tpu as pltpu
import jax.experimental.pallas.tpu_sc as plsc

NC = 2
NS = 16
NW = NC * NS
CH = 128

_f32 = jnp.float32


def _sc_mesh():
    return plsc.VectorSubcoreMesh(
        core_axis_name="c", subcore_axis_name="s", num_cores=NC, num_subcores=NS
    )


def _sc_deg(idxc, ones8, zeros8, n_pad):
    nch = idxc.shape[0] // NW
    assert nch % 2 == 0
    rps = n_pad // NS
    rch = rps // CH

    def body(idx_hbm, ones_hbm, zeros_hbm, out_hbm,
             idx0, idx1, ones_v, z8_v, is0, is1, acc):
        c = lax.axis_index("c")
        s = lax.axis_index("s")
        wid = s * NC + c
        base = wid * nch
        row0 = s * rps
        pltpu.sync_copy(ones_hbm, ones_v)
        pltpu.sync_copy(zeros_hbm, z8_v)
        for t in range(rch):
            pltpu.sync_copy(z8_v, acc.at[pl.ds(row0 + t * CH, CH)])
        plsc.subcore_barrier()

        idxb = (idx0, idx1)
        isem = (is0, is1)

        def fetch(j, slot):
            pltpu.async_copy(idx_hbm.at[base + j], idxb[slot], isem[slot])

        def wait_fetch(j, slot):
            pltpu.make_async_copy(idx_hbm.at[base + j], idxb[slot],
                                  isem[slot]).wait()

        def scat(slot):
            pltpu.sync_copy(ones_v, acc.at[idxb[slot].at[1]], add=True)

        fetch(0, 0)
        fetch(1, 1)

        def step(p, carry):
            j = p * 2
            wait_fetch(j, 0)
            scat(0)
            fetch(j + 2, 0)
            wait_fetch(j + 1, 1)
            scat(1)
            fetch(j + 3, 1)
            return carry

        lax.fori_loop(0, nch // 2 - 1, step, 0)
        wait_fetch(nch - 2, 0)
        scat(0)
        wait_fetch(nch - 1, 1)
        scat(1)

        plsc.subcore_barrier()
        for t in range(rch):
            sl = pl.ds(row0 + t * CH, CH)
            pltpu.sync_copy(acc.at[sl], z8_v)
            pltpu.sync_copy(z8_v, out_hbm.at[c, sl])

    f = pl.kernel(
        body,
        out_type=jax.ShapeDtypeStruct((NC, n_pad, 8), _f32),
        mesh=_sc_mesh(),
        scratch_types=[
            pltpu.VMEM((2, CH), jnp.int32),
            pltpu.VMEM((2, CH), jnp.int32),
            pltpu.VMEM((CH, 8), _f32),
            pltpu.VMEM((CH, 8), _f32),
            pltpu.SemaphoreType.DMA,
            pltpu.SemaphoreType.DMA,
            pltpu.VMEM_SHARED((n_pad, 8), _f32),
        ],
        compiler_params=pltpu.CompilerParams(use_tc_tiling_on_sc=False),
    )
    return f(idxc, ones8, zeros8)


def _sc_agg(table, idxc, zrows, n_pad, split0):
    n, d = table.shape
    pairs = idxc.shape[0] // NS
    nch0 = max(2, int(pairs * split0) // 2 * 2)
    nch1 = pairs - nch0
    assert nch1 >= 2
    rps = n_pad // NS
    rch = rps // CH

    def body(table_hbm, idx_hbm, zeros_hbm, out_hbm,
             idx0, idx1, rows0, rows1, sem0, sem1, acc):
        c = lax.axis_index("c")
        s = lax.axis_index("s")
        row0 = s * rps
        pltpu.sync_copy(zeros_hbm, rows0)
        for t in range(rch):
            pltpu.sync_copy(rows0, acc.at[pl.ds(row0 + t * CH, CH)])
        plsc.subcore_barrier()

        rows = (rows0, rows1)
        idxb = (idx0, idx1)
        sems = (sem0, sem1)

        def make_pipeline(nch, base):
            def fetch_start(j, slot):
                pltpu.sync_copy(idx_hbm.at[base + j], idxb[slot])
                pltpu.async_copy(table_hbm.at[idxb[slot].at[0]], rows[slot],
                                 sems[slot])

            def finish(slot):
                pltpu.make_async_copy(table_hbm.at[idxb[slot].at[0]],
                                      rows[slot], sems[slot]).wait()
                pltpu.sync_copy(rows[slot], acc.at[idxb[slot].at[1]], add=True)

            def run():
                fetch_start(0, 0)

                def step(p, carry):
                    j = p * 2
                    fetch_start(j + 1, 1)
                    finish(0)
                    fetch_start(j + 2, 0)
                    finish(1)
                    return carry

                lax.fori_loop(0, nch // 2 - 1, step, 0)
                fetch_start(nch - 1, 1)
                finish(0)
                finish(1)

            return run

        run0 = make_pipeline(nch0, s * nch0)
        run1 = make_pipeline(nch1, NS * nch0 + s * nch1)
        lax.cond(c == 0, run0, run1)

        plsc.subcore_barrier()
        for t in range(rch):
            sl = pl.ds(row0 + t * CH, CH)
            pltpu.sync_copy(acc.at[sl], rows0)
            pltpu.sync_copy(rows0, out_hbm.at[c, sl])

    f = pl.kernel(
        body,
        out_type=jax.ShapeDtypeStruct((NC, n_pad, d), _f32),
        mesh=_sc_mesh(),
        scratch_types=[
            pltpu.VMEM((2, CH), jnp.int32),
            pltpu.VMEM((2, CH), jnp.int32),
            pltpu.VMEM((CH, d), _f32),
            pltpu.VMEM((CH, d), _f32),
            pltpu.SemaphoreType.DMA,
            pltpu.SemaphoreType.DMA,
            pltpu.VMEM_SHARED((n_pad, d), _f32),
        ],
        compiler_params=pltpu.CompilerParams(use_tc_tiling_on_sc=False),
    )
    return f(table, idxc, zrows)


def _block(n):
    for b in (1000, 500, 250, 200, 125, 100, 50, 25, 10, 5):
        if n % b == 0:
            return b
    return n


def _tc_l1(x, w1t, degp):
    n, dx = x.shape
    dd = w1t.shape[1]
    bs = _block(n)

    def body(x_ref, w_ref, d0_ref, d1_ref, hp_ref, dinv_ref):
        deg = d0_ref[0, :, 0:1] + d1_ref[0, :, 0:1] + 1.0
        dinv = lax.rsqrt(deg)
        hp_ref[...] = (
            jnp.dot(x_ref[...], w_ref[...], preferred_element_type=_f32) * dinv
        )
        dinv_ref[...] = jnp.broadcast_to(dinv, dinv_ref.shape)

    return pl.pallas_call(
        body,
        grid=(n // bs,),
        in_specs=[
            pl.BlockSpec((bs, dx), lambda i: (i, 0)),
            pl.BlockSpec((dx, dd), lambda i: (0, 0)),
            pl.BlockSpec((1, bs, 8), lambda i: (0, i, 0)),
            pl.BlockSpec((1, bs, 8), lambda i: (1, i, 0)),
        ],
        out_specs=[
            pl.BlockSpec((bs, dd), lambda i: (i, 0)),
            pl.BlockSpec((bs, 8), lambda i: (i, 0)),
        ],
        out_shape=[
            jax.ShapeDtypeStruct((n, dd), _f32),
            jax.ShapeDtypeStruct((n, 8), _f32),
        ],
    )(x, w1t, degp, degp)


def _tc_combine(parts, hp, dinv8, b):
    n, d = hp.shape
    bs = _block(n)

    def body(p0_ref, p1_ref, hp_ref, dinv_ref, b_ref, z_ref, st_ref):
        i = pl.program_id(0)
        z = (dinv_ref[:, 0:1] * (p0_ref[0] + p1_ref[0] + hp_ref[...])
             + b_ref[...])
        z_ref[...] = z
        part = jnp.concatenate(
            [jnp.sum(z, 0, keepdims=True), jnp.sum(z * z, 0, keepdims=True)], 0
        )

        @pl.when(i == 0)
        def _init():
            st_ref[...] = part

        @pl.when(i > 0)
        def _acc():
            st_ref[...] += part

    return pl.pallas_call(
        body,
        grid=(n // bs,),
        in_specs=[
            pl.BlockSpec((1, bs, d), lambda i: (0, i, 0)),
            pl.BlockSpec((1, bs, d), lambda i: (1, i, 0)),
            pl.BlockSpec((bs, d), lambda i: (i, 0)),
            pl.BlockSpec((bs, 8), lambda i: (i, 0)),
            pl.BlockSpec((1, d), lambda i: (0, 0)),
        ],
        out_specs=[
            pl.BlockSpec((bs, d), lambda i: (i, 0)),
            pl.BlockSpec((2, d), lambda i: (0, 0)),
        ],
        out_shape=[
            jax.ShapeDtypeStruct((n, d), _f32),
            jax.ShapeDtypeStruct((2, d), _f32),
        ],
    )(parts, parts, hp, dinv8, b)


def _tc_bnmm(z, st, g, beta, wt, dinv8=None, bias=None):
    n, d = z.shape
    do = wt.shape[1]
    bs = _block(n)
    nf = float(n)

    def body(*refs):
        if dinv8 is not None:
            z_ref, st_ref, g_ref, beta_ref, w_ref, dv_ref, o_ref = refs
        elif bias is not None:
            z_ref, st_ref, g_ref, beta_ref, w_ref, b_ref, o_ref = refs
        else:
            z_ref, st_ref, g_ref, beta_ref, w_ref, o_ref = refs
        mean = st_ref[0:1, :] / nf
        var = st_ref[1:2, :] / nf - mean * mean
        sc = g_ref[...] * lax.rsqrt(var + 1e-5)
        h = jnp.maximum(z_ref[...] * sc + (beta_ref[...] - mean * sc), 0.0)
        o = jnp.dot(h, w_ref[...], preferred_element_type=_f32)
        if dinv8 is not None:
            o = o * dv_ref[:, 0:1]
        if bias is not None:
            o = o + b_ref[...]
        o_ref[...] = o

    in_specs = [
        pl.BlockSpec((bs, d), lambda i: (i, 0)),
        pl.BlockSpec((2, d), lambda i: (0, 0)),
        pl.BlockSpec((1, d), lambda i: (0, 0)),
        pl.BlockSpec((1, d), lambda i: (0, 0)),
        pl.BlockSpec((d, do), lambda i: (0, 0)),
    ]
    args = [z, st, g, beta, wt]
    if dinv8 is not None:
        in_specs.append(pl.BlockSpec((bs, 8), lambda i: (i, 0)))
        args.append(dinv8)
    if bias is not None:
        in_specs.append(pl.BlockSpec((1, do), lambda i: (0, 0)))
        args.append(bias)

    return pl.pallas_call(
        body,
        grid=(n // bs,),
        in_specs=in_specs,
        out_specs=pl.BlockSpec((bs, do), lambda i: (i, 0)),
        out_shape=jax.ShapeDtypeStruct((n, do), _f32),
    )(*args)


def _ceil_to(a, m):
    return (a + m - 1) // m * m


def kernel(x, edge_index, W1, b1, g1, beta1, W2, b2, g2, beta2, Wfc, bfc):
    n = x.shape[0]
    e = edge_index.shape[1]
    n_pad = _ceil_to(n + 1, NS * CH)
    e_pad = _ceil_to(e, NW * CH * 2)

    src_p = jnp.concatenate(
        [edge_index[0], jnp.zeros((e_pad - e,), edge_index.dtype)]
    )
    dst_p = jnp.concatenate(
        [edge_index[1], jnp.full((e_pad - e,), n, edge_index.dtype)]
    )

    idxc = jnp.stack([src_p.reshape(-1, CH), dst_p.reshape(-1, CH)], axis=1)

    ones8 = jnp.ones((CH, 8), _f32)
    zeros8 = jnp.zeros((CH, 8), _f32)
    degp = _sc_deg(idxc, ones8, zeros8, n_pad)

    hp1, dinv8 = _tc_l1(x, W1.T, degp)

    parts1 = _sc_agg(hp1, idxc, jnp.zeros((CH, hp1.shape[1]), _f32), n_pad, 0.80)
    z1, st1 = _tc_combine(parts1, hp1, dinv8, b1.reshape(1, -1))
    hp2 = _tc_bnmm(z1, st1, g1.reshape(1, -1), beta1.reshape(1, -1), W2.T, dinv8=dinv8)

    parts2 = _sc_agg(hp2, idxc, jnp.zeros((CH, hp2.shape[1]), _f32), n_pad, 0.68)
    z2, st2 = _tc_combine(parts2, hp2, dinv8, b2.reshape(1, -1))
    return _tc_bnmm(z2, st2, g2.reshape(1, -1), beta2.reshape(1, -1), Wfc.T, bias=bfc.reshape(1, -1))


# speedup vs baseline: 2.2112x; 1.0264x over previous
import jax
import jax.numpy as jnp
from jax import lax
from jax.experimental import pallas as pl
from jax.experimental.pallas import tpu as pltpu
import jax.experimental.pallas.tpu_sc as plsc

NC = 2
NS = 16
NW = NC * NS
CH = 128

_f32 = jnp.float32


def _sc_mesh():
    return plsc.VectorSubcoreMesh(
        core_axis_name="c", subcore_axis_name="s", num_cores=NC, num_subcores=NS
    )


def _sc_deg(idxc, ones8, zeros8, n_pad):
    nch = idxc.shape[0] // NW
    assert nch % 2 == 0
    rps = n_pad // NS
    rch = rps // CH

    def body(idx_hbm, ones_hbm, zeros_hbm, out_hbm,
             idx0, idx1, ones_v, z8_v, is0, is1, acc):
        c = lax.axis_index("c")
        s = lax.axis_index("s")
        wid = s * NC + c
        base = wid * nch
        row0 = s * rps
        pltpu.sync_copy(ones_hbm, ones_v)
        pltpu.sync_copy(zeros_hbm, z8_v)
        for t in range(rch):
            pltpu.sync_copy(z8_v, acc.at[pl.ds(row0 + t * CH, CH)])
        plsc.subcore_barrier()

        idxb = (idx0, idx1)
        isem = (is0, is1)

        def fetch(j, slot):
            pltpu.async_copy(idx_hbm.at[base + j], idxb[slot], isem[slot])

        def wait_fetch(j, slot):
            pltpu.make_async_copy(idx_hbm.at[base + j], idxb[slot],
                                  isem[slot]).wait()

        def scat(slot):
            pltpu.sync_copy(ones_v, acc.at[idxb[slot].at[1]], add=True)

        fetch(0, 0)
        fetch(1, 1)

        def step(p, carry):
            j = p * 2
            wait_fetch(j, 0)
            scat(0)
            fetch(j + 2, 0)
            wait_fetch(j + 1, 1)
            scat(1)
            fetch(j + 3, 1)
            return carry

        lax.fori_loop(0, nch // 2 - 1, step, 0)
        wait_fetch(nch - 2, 0)
        scat(0)
        wait_fetch(nch - 1, 1)
        scat(1)

        plsc.subcore_barrier()
        for t in range(rch):
            sl = pl.ds(row0 + t * CH, CH)
            pltpu.sync_copy(acc.at[sl], z8_v)
            pltpu.sync_copy(z8_v, out_hbm.at[c, sl])

    f = pl.kernel(
        body,
        out_type=jax.ShapeDtypeStruct((NC, n_pad, 8), _f32),
        mesh=_sc_mesh(),
        scratch_types=[
            pltpu.VMEM((2, CH), jnp.int32),
            pltpu.VMEM((2, CH), jnp.int32),
            pltpu.VMEM((CH, 8), _f32),
            pltpu.VMEM((CH, 8), _f32),
            pltpu.SemaphoreType.DMA,
            pltpu.SemaphoreType.DMA,
            pltpu.VMEM_SHARED((n_pad, 8), _f32),
        ],
        compiler_params=pltpu.CompilerParams(use_tc_tiling_on_sc=False),
    )
    return f(idxc, ones8, zeros8)


def _sc_agg(table, idxc, zrows, n_pad, split0):
    n, d = table.shape
    pairs = idxc.shape[0] // NS
    nch0 = max(2, int(pairs * split0) // 2 * 2)
    nch1 = pairs - nch0
    assert nch1 >= 2
    rps = n_pad // NS
    rch = rps // CH

    def body(table_hbm, idx_hbm, zeros_hbm, out_hbm,
             idx0, idx1, rows0, rows1, sem0, sem1, acc):
        c = lax.axis_index("c")
        s = lax.axis_index("s")
        row0 = s * rps
        pltpu.sync_copy(zeros_hbm, rows0)
        for t in range(rch):
            pltpu.sync_copy(rows0, acc.at[pl.ds(row0 + t * CH, CH)])
        plsc.subcore_barrier()

        rows = (rows0, rows1)
        idxb = (idx0, idx1)
        sems = (sem0, sem1)

        def make_pipeline(nch, base):
            def fetch_start(j, slot):
                pltpu.sync_copy(idx_hbm.at[base + j], idxb[slot])
                pltpu.async_copy(table_hbm.at[idxb[slot].at[0]], rows[slot],
                                 sems[slot])

            def finish(slot):
                pltpu.make_async_copy(table_hbm.at[idxb[slot].at[0]],
                                      rows[slot], sems[slot]).wait()
                pltpu.sync_copy(rows[slot], acc.at[idxb[slot].at[1]], add=True)

            def run():
                fetch_start(0, 0)

                def step(p, carry):
                    j = p * 2
                    fetch_start(j + 1, 1)
                    finish(0)
                    fetch_start(j + 2, 0)
                    finish(1)
                    return carry

                lax.fori_loop(0, nch // 2 - 1, step, 0)
                fetch_start(nch - 1, 1)
                finish(0)
                finish(1)

            return run

        run0 = make_pipeline(nch0, s * nch0)
        run1 = make_pipeline(nch1, NS * nch0 + s * nch1)
        lax.cond(c == 0, run0, run1)

        plsc.subcore_barrier()
        for t in range(rch):
            sl = pl.ds(row0 + t * CH, CH)
            pltpu.sync_copy(acc.at[sl], rows0)
            pltpu.sync_copy(rows0, out_hbm.at[c, sl])

    f = pl.kernel(
        body,
        out_type=jax.ShapeDtypeStruct((NC, n_pad, d), _f32),
        mesh=_sc_mesh(),
        scratch_types=[
            pltpu.VMEM((2, CH), jnp.int32),
            pltpu.VMEM((2, CH), jnp.int32),
            pltpu.VMEM((CH, d), _f32),
            pltpu.VMEM((CH, d), _f32),
            pltpu.SemaphoreType.DMA,
            pltpu.SemaphoreType.DMA,
            pltpu.VMEM_SHARED((n_pad, d), _f32),
        ],
        compiler_params=pltpu.CompilerParams(use_tc_tiling_on_sc=False),
    )
    return f(table, idxc, zrows)


def _block(n):
    for b in (1000, 500, 250, 200, 125, 100, 50, 25, 10, 5):
        if n % b == 0:
            return b
    return n


def _tc_l1(x, w1t, degp):
    n, dx = x.shape
    dd = w1t.shape[1]
    bs = _block(n)

    def body(x_ref, w_ref, d0_ref, d1_ref, hp_ref, dinv_ref):
        deg = d0_ref[0, :, 0:1] + d1_ref[0, :, 0:1] + 1.0
        dinv = lax.rsqrt(deg)
        hp_ref[...] = (
            jnp.dot(x_ref[...], w_ref[...], preferred_element_type=_f32) * dinv
        )
        dinv_ref[...] = jnp.broadcast_to(dinv, dinv_ref.shape)

    return pl.pallas_call(
        body,
        grid=(n // bs,),
        in_specs=[
            pl.BlockSpec((bs, dx), lambda i: (i, 0)),
            pl.BlockSpec((dx, dd), lambda i: (0, 0)),
            pl.BlockSpec((1, bs, 8), lambda i: (0, i, 0)),
            pl.BlockSpec((1, bs, 8), lambda i: (1, i, 0)),
        ],
        out_specs=[
            pl.BlockSpec((bs, dd), lambda i: (i, 0)),
            pl.BlockSpec((bs, 8), lambda i: (i, 0)),
        ],
        out_shape=[
            jax.ShapeDtypeStruct((n, dd), _f32),
            jax.ShapeDtypeStruct((n, 8), _f32),
        ],
    )(x, w1t, degp, degp)


def _tc_combine(parts, hp, dinv8, b):
    n, d = hp.shape
    bs = _block(n)

    def body(p0_ref, p1_ref, hp_ref, dinv_ref, b_ref, z_ref, st_ref):
        i = pl.program_id(0)
        z = (dinv_ref[:, 0:1] * (p0_ref[0] + p1_ref[0] + hp_ref[...])
             + b_ref[...])
        z_ref[...] = z
        part = jnp.concatenate(
            [jnp.sum(z, 0, keepdims=True), jnp.sum(z * z, 0, keepdims=True)], 0
        )

        @pl.when(i == 0)
        def _init():
            st_ref[...] = part

        @pl.when(i > 0)
        def _acc():
            st_ref[...] += part

    return pl.pallas_call(
        body,
        grid=(n // bs,),
        in_specs=[
            pl.BlockSpec((1, bs, d), lambda i: (0, i, 0)),
            pl.BlockSpec((1, bs, d), lambda i: (1, i, 0)),
            pl.BlockSpec((bs, d), lambda i: (i, 0)),
            pl.BlockSpec((bs, 8), lambda i: (i, 0)),
            pl.BlockSpec((1, d), lambda i: (0, 0)),
        ],
        out_specs=[
            pl.BlockSpec((bs, d), lambda i: (i, 0)),
            pl.BlockSpec((2, d), lambda i: (0, 0)),
        ],
        out_shape=[
            jax.ShapeDtypeStruct((n, d), _f32),
            jax.ShapeDtypeStruct((2, d), _f32),
        ],
    )(parts, parts, hp, dinv8, b)


def _tc_bnmm(z, st, g, beta, wt, dinv8=None, bias=None):
    n, d = z.shape
    do = wt.shape[1]
    bs = _block(n)
    nf = float(n)

    def body(*refs):
        if dinv8 is not None:
            z_ref, st_ref, g_ref, beta_ref, w_ref, dv_ref, o_ref = refs
        elif bias is not None:
            z_ref, st_ref, g_ref, beta_ref, w_ref, b_ref, o_ref = refs
        else:
            z_ref, st_ref, g_ref, beta_ref, w_ref, o_ref = refs
        mean = st_ref[0:1, :] / nf
        var = st_ref[1:2, :] / nf - mean * mean
        sc = g_ref[...] * lax.rsqrt(var + 1e-5)
        h = jnp.maximum(z_ref[...] * sc + (beta_ref[...] - mean * sc), 0.0)
        o = jnp.dot(h, w_ref[...], preferred_element_type=_f32)
        if dinv8 is not None:
            o = o * dv_ref[:, 0:1]
        if bias is not None:
            o = o + b_ref[...]
        o_ref[...] = o

    in_specs = [
        pl.BlockSpec((bs, d), lambda i: (i, 0)),
        pl.BlockSpec((2, d), lambda i: (0, 0)),
        pl.BlockSpec((1, d), lambda i: (0, 0)),
        pl.BlockSpec((1, d), lambda i: (0, 0)),
        pl.BlockSpec((d, do), lambda i: (0, 0)),
    ]
    args = [z, st, g, beta, wt]
    if dinv8 is not None:
        in_specs.append(pl.BlockSpec((bs, 8), lambda i: (i, 0)))
        args.append(dinv8)
    if bias is not None:
        in_specs.append(pl.BlockSpec((1, do), lambda i: (0, 0)))
        args.append(bias)

    return pl.pallas_call(
        body,
        grid=(n // bs,),
        in_specs=in_specs,
        out_specs=pl.BlockSpec((bs, do), lambda i: (i, 0)),
        out_shape=jax.ShapeDtypeStruct((n, do), _f32),
    )(*args)


def _ceil_to(a, m):
    return (a + m - 1) // m * m


def kernel(x, edge_index, W1, b1, g1, beta1, W2, b2, g2, beta2, Wfc, bfc):
    n = x.shape[0]
    e = edge_index.shape[1]
    n_pad = _ceil_to(n + 1, NS * CH)
    e_pad = _ceil_to(e, NW * CH * 2)

    src_p = jnp.concatenate(
        [edge_index[0], jnp.zeros((e_pad - e,), edge_index.dtype)]
    )
    dst_p = jnp.concatenate(
        [edge_index[1], jnp.full((e_pad - e,), n, edge_index.dtype)]
    )

    idxc = jnp.stack([src_p.reshape(-1, CH), dst_p.reshape(-1, CH)], axis=1)

    ones8 = jnp.ones((CH, 8), _f32)
    zeros8 = jnp.zeros((CH, 8), _f32)
    degp = _sc_deg(idxc, ones8, zeros8, n_pad)

    hp1, dinv8 = _tc_l1(x, W1.T, degp)

    parts1 = _sc_agg(hp1, idxc, jnp.zeros((CH, hp1.shape[1]), _f32), n_pad, 0.85)
    z1, st1 = _tc_combine(parts1, hp1, dinv8, b1.reshape(1, -1))
    hp2 = _tc_bnmm(z1, st1, g1.reshape(1, -1), beta1.reshape(1, -1), W2.T, dinv8=dinv8)

    parts2 = _sc_agg(hp2, idxc, jnp.zeros((CH, hp2.shape[1]), _f32), n_pad, 0.72)
    z2, st2 = _tc_combine(parts2, hp2, dinv8, b2.reshape(1, -1))
    return _tc_bnmm(z2, st2, g2.reshape(1, -1), beta2.reshape(1, -1), Wfc.T, bias=bfc.reshape(1, -1))


# speedup vs baseline: 2.2119x; 1.0003x over previous
import jax
import jax.numpy as jnp
from jax import lax
from jax.experimental import pallas as pl
from jax.experimental.pallas import tpu as pltpu
import jax.experimental.pallas.tpu_sc as plsc

NC = 2
NS = 16
NW = NC * NS
CH = 128

_f32 = jnp.float32


def _sc_mesh():
    return plsc.VectorSubcoreMesh(
        core_axis_name="c", subcore_axis_name="s", num_cores=NC, num_subcores=NS
    )


def _sc_deg(idxc, ones8, zeros8, n_pad):
    nch = idxc.shape[0] // NW
    assert nch % 2 == 0
    rps = n_pad // NS
    rch = rps // CH

    def body(idx_hbm, ones_hbm, zeros_hbm, out_hbm,
             idx0, idx1, ones_v, z8_v, is0, is1, acc):
        c = lax.axis_index("c")
        s = lax.axis_index("s")
        wid = s * NC + c
        base = wid * nch
        row0 = s * rps
        pltpu.sync_copy(ones_hbm, ones_v)
        pltpu.sync_copy(zeros_hbm, z8_v)
        for t in range(rch):
            pltpu.sync_copy(z8_v, acc.at[pl.ds(row0 + t * CH, CH)])
        plsc.subcore_barrier()

        idxb = (idx0, idx1)
        isem = (is0, is1)

        def fetch(j, slot):
            pltpu.async_copy(idx_hbm.at[base + j], idxb[slot], isem[slot])

        def wait_fetch(j, slot):
            pltpu.make_async_copy(idx_hbm.at[base + j], idxb[slot],
                                  isem[slot]).wait()

        def scat(slot):
            pltpu.sync_copy(ones_v, acc.at[idxb[slot].at[1]], add=True)

        fetch(0, 0)
        fetch(1, 1)

        def step(p, carry):
            j = p * 2
            wait_fetch(j, 0)
            scat(0)
            fetch(j + 2, 0)
            wait_fetch(j + 1, 1)
            scat(1)
            fetch(j + 3, 1)
            return carry

        lax.fori_loop(0, nch // 2 - 1, step, 0)
        wait_fetch(nch - 2, 0)
        scat(0)
        wait_fetch(nch - 1, 1)
        scat(1)

        plsc.subcore_barrier()
        for t in range(rch):
            sl = pl.ds(row0 + t * CH, CH)
            pltpu.sync_copy(acc.at[sl], z8_v)
            pltpu.sync_copy(z8_v, out_hbm.at[c, sl])

    f = pl.kernel(
        body,
        out_type=jax.ShapeDtypeStruct((NC, n_pad, 8), _f32),
        mesh=_sc_mesh(),
        scratch_types=[
            pltpu.VMEM((2, CH), jnp.int32),
            pltpu.VMEM((2, CH), jnp.int32),
            pltpu.VMEM((CH, 8), _f32),
            pltpu.VMEM((CH, 8), _f32),
            pltpu.SemaphoreType.DMA,
            pltpu.SemaphoreType.DMA,
            pltpu.VMEM_SHARED((n_pad, 8), _f32),
        ],
        compiler_params=pltpu.CompilerParams(use_tc_tiling_on_sc=False),
    )
    return f(idxc, ones8, zeros8)


def _sc_agg(table, idxc, zrows, n_pad, split0):
    n, d = table.shape
    pairs = idxc.shape[0] // NS
    nch0 = max(2, int(pairs * split0) // 2 * 2)
    nch1 = pairs - nch0
    assert nch1 >= 2
    rps = n_pad // NS
    rch = rps // CH

    def body(table_hbm, idx_hbm, zeros_hbm, out_hbm,
             idx0, idx1, rows0, rows1, sem0, sem1, acc):
        c = lax.axis_index("c")
        s = lax.axis_index("s")
        row0 = s * rps
        pltpu.sync_copy(zeros_hbm, rows0)
        for t in range(rch):
            pltpu.sync_copy(rows0, acc.at[pl.ds(row0 + t * CH, CH)])
        plsc.subcore_barrier()

        rows = (rows0, rows1)
        idxb = (idx0, idx1)
        sems = (sem0, sem1)

        def make_pipeline(nch, base):
            def fetch_start(j, slot):
                pltpu.sync_copy(idx_hbm.at[base + j], idxb[slot])
                pltpu.async_copy(table_hbm.at[idxb[slot].at[0]], rows[slot],
                                 sems[slot])

            def finish(slot):
                pltpu.make_async_copy(table_hbm.at[idxb[slot].at[0]],
                                      rows[slot], sems[slot]).wait()
                pltpu.sync_copy(rows[slot], acc.at[idxb[slot].at[1]], add=True)

            def run():
                fetch_start(0, 0)

                def step(p, carry):
                    j = p * 2
                    fetch_start(j + 1, 1)
                    finish(0)
                    fetch_start(j + 2, 0)
                    finish(1)
                    return carry

                lax.fori_loop(0, nch // 2 - 1, step, 0)
                fetch_start(nch - 1, 1)
                finish(0)
                finish(1)

            return run

        run0 = make_pipeline(nch0, s * nch0)
        run1 = make_pipeline(nch1, NS * nch0 + s * nch1)
        lax.cond(c == 0, run0, run1)

        plsc.subcore_barrier()
        for t in range(rch):
            sl = pl.ds(row0 + t * CH, CH)
            pltpu.sync_copy(acc.at[sl], rows0)
            pltpu.sync_copy(rows0, out_hbm.at[c, sl])

    f = pl.kernel(
        body,
        out_type=jax.ShapeDtypeStruct((NC, n_pad, d), _f32),
        mesh=_sc_mesh(),
        scratch_types=[
            pltpu.VMEM((2, CH), jnp.int32),
            pltpu.VMEM((2, CH), jnp.int32),
            pltpu.VMEM((CH, d), _f32),
            pltpu.VMEM((CH, d), _f32),
            pltpu.SemaphoreType.DMA,
            pltpu.SemaphoreType.DMA,
            pltpu.VMEM_SHARED((n_pad, d), _f32),
        ],
        compiler_params=pltpu.CompilerParams(use_tc_tiling_on_sc=False),
    )
    return f(table, idxc, zrows)


def _block(n):
    for b in (1000, 500, 250, 200, 125, 100, 50, 25, 10, 5):
        if n % b == 0:
            return b
    return n


def _tc_l1(x, w1t, degp):
    n, dx = x.shape
    dd = w1t.shape[1]
    bs = _block(n)

    def body(x_ref, w_ref, d0_ref, d1_ref, hp_ref, dinv_ref):
        deg = d0_ref[0, :, 0:1] + d1_ref[0, :, 0:1] + 1.0
        dinv = lax.rsqrt(deg)
        hp_ref[...] = (
            jnp.dot(x_ref[...], w_ref[...], preferred_element_type=_f32) * dinv
        )
        dinv_ref[...] = jnp.broadcast_to(dinv, dinv_ref.shape)

    return pl.pallas_call(
        body,
        grid=(n // bs,),
        in_specs=[
            pl.BlockSpec((bs, dx), lambda i: (i, 0)),
            pl.BlockSpec((dx, dd), lambda i: (0, 0)),
            pl.BlockSpec((1, bs, 8), lambda i: (0, i, 0)),
            pl.BlockSpec((1, bs, 8), lambda i: (1, i, 0)),
        ],
        out_specs=[
            pl.BlockSpec((bs, dd), lambda i: (i, 0)),
            pl.BlockSpec((bs, 8), lambda i: (i, 0)),
        ],
        out_shape=[
            jax.ShapeDtypeStruct((n, dd), _f32),
            jax.ShapeDtypeStruct((n, 8), _f32),
        ],
    )(x, w1t, degp, degp)


def _tc_combine(parts, hp, dinv8, b):
    n, d = hp.shape
    bs = _block(n)

    def body(p0_ref, p1_ref, hp_ref, dinv_ref, b_ref, z_ref, st_ref):
        i = pl.program_id(0)
        z = (dinv_ref[:, 0:1] * (p0_ref[0] + p1_ref[0] + hp_ref[...])
             + b_ref[...])
        z_ref[...] = z
        part = jnp.concatenate(
            [jnp.sum(z, 0, keepdims=True), jnp.sum(z * z, 0, keepdims=True)], 0
        )

        @pl.when(i == 0)
        def _init():
            st_ref[...] = part

        @pl.when(i > 0)
        def _acc():
            st_ref[...] += part

    return pl.pallas_call(
        body,
        grid=(n // bs,),
        in_specs=[
            pl.BlockSpec((1, bs, d), lambda i: (0, i, 0)),
            pl.BlockSpec((1, bs, d), lambda i: (1, i, 0)),
            pl.BlockSpec((bs, d), lambda i: (i, 0)),
            pl.BlockSpec((bs, 8), lambda i: (i, 0)),
            pl.BlockSpec((1, d), lambda i: (0, 0)),
        ],
        out_specs=[
            pl.BlockSpec((bs, d), lambda i: (i, 0)),
            pl.BlockSpec((2, d), lambda i: (0, 0)),
        ],
        out_shape=[
            jax.ShapeDtypeStruct((n, d), _f32),
            jax.ShapeDtypeStruct((2, d), _f32),
        ],
    )(parts, parts, hp, dinv8, b)


def _tc_bnmm(z, st, g, beta, wt, dinv8=None, bias=None):
    n, d = z.shape
    do = wt.shape[1]
    bs = _block(n)
    nf = float(n)

    def body(*refs):
        if dinv8 is not None:
            z_ref, st_ref, g_ref, beta_ref, w_ref, dv_ref, o_ref = refs
        elif bias is not None:
            z_ref, st_ref, g_ref, beta_ref, w_ref, b_ref, o_ref = refs
        else:
            z_ref, st_ref, g_ref, beta_ref, w_ref, o_ref = refs
        mean = st_ref[0:1, :] / nf
        var = st_ref[1:2, :] / nf - mean * mean
        sc = g_ref[...] * lax.rsqrt(var + 1e-5)
        h = jnp.maximum(z_ref[...] * sc + (beta_ref[...] - mean * sc), 0.0)
        o = jnp.dot(h, w_ref[...], preferred_element_type=_f32)
        if dinv8 is not None:
            o = o * dv_ref[:, 0:1]
        if bias is not None:
            o = o + b_ref[...]
        o_ref[...] = o

    in_specs = [
        pl.BlockSpec((bs, d), lambda i: (i, 0)),
        pl.BlockSpec((2, d), lambda i: (0, 0)),
        pl.BlockSpec((1, d), lambda i: (0, 0)),
        pl.BlockSpec((1, d), lambda i: (0, 0)),
        pl.BlockSpec((d, do), lambda i: (0, 0)),
    ]
    args = [z, st, g, beta, wt]
    if dinv8 is not None:
        in_specs.append(pl.BlockSpec((bs, 8), lambda i: (i, 0)))
        args.append(dinv8)
    if bias is not None:
        in_specs.append(pl.BlockSpec((1, do), lambda i: (0, 0)))
        args.append(bias)

    return pl.pallas_call(
        body,
        grid=(n // bs,),
        in_specs=in_specs,
        out_specs=pl.BlockSpec((bs, do), lambda i: (i, 0)),
        out_shape=jax.ShapeDtypeStruct((n, do), _f32),
    )(*args)


def _ceil_to(a, m):
    return (a + m - 1) // m * m


def kernel(x, edge_index, W1, b1, g1, beta1, W2, b2, g2, beta2, Wfc, bfc):
    n = x.shape[0]
    e = edge_index.shape[1]
    n_pad = _ceil_to(n + 1, NS * CH)
    e_pad = _ceil_to(e, NW * CH * 2)

    src_p = jnp.concatenate(
        [edge_index[0], jnp.zeros((e_pad - e,), edge_index.dtype)]
    )
    dst_p = jnp.concatenate(
        [edge_index[1], jnp.full((e_pad - e,), n, edge_index.dtype)]
    )

    idxc = jnp.stack([src_p.reshape(-1, CH), dst_p.reshape(-1, CH)], axis=1)

    ones8 = jnp.ones((CH, 8), _f32)
    zeros8 = jnp.zeros((CH, 8), _f32)
    degp = _sc_deg(idxc, ones8, zeros8, n_pad)

    hp1, dinv8 = _tc_l1(x, W1.T, degp)

    parts1 = _sc_agg(hp1, idxc, jnp.zeros((CH, hp1.shape[1]), _f32), n_pad, 0.90)
    z1, st1 = _tc_combine(parts1, hp1, dinv8, b1.reshape(1, -1))
    hp2 = _tc_bnmm(z1, st1, g1.reshape(1, -1), beta1.reshape(1, -1), W2.T, dinv8=dinv8)

    parts2 = _sc_agg(hp2, idxc, jnp.zeros((CH, hp2.shape[1]), _f32), n_pad, 0.76)
    z2, st2 = _tc_combine(parts2, hp2, dinv8, b2.reshape(1, -1))
    return _tc_bnmm(z2, st2, g2.reshape(1, -1), beta2.reshape(1, -1), Wfc.T, bias=bfc.reshape(1, -1))


# speedup vs baseline: 2.2248x; 1.0059x over previous
import jax
import jax.numpy as jnp
from jax import lax
from jax.experimental import pallas as pl
from jax.experimental.pallas import tpu as pltpu
import jax.experimental.pallas.tpu_sc as plsc

NC = 2
NS = 16
NW = NC * NS
CH = 128

_f32 = jnp.float32


def _sc_mesh():
    return plsc.VectorSubcoreMesh(
        core_axis_name="c", subcore_axis_name="s", num_cores=NC, num_subcores=NS
    )


def _sc_deg(idxc, ones8, zeros8, n_pad):
    nch = idxc.shape[0] // NW
    assert nch % 2 == 0
    rps = n_pad // NS
    rch = rps // CH

    def body(idx_hbm, ones_hbm, zeros_hbm, out_hbm,
             idx0, idx1, ones_v, z8_v, is0, is1, acc):
        c = lax.axis_index("c")
        s = lax.axis_index("s")
        wid = s * NC + c
        base = wid * nch
        row0 = s * rps
        pltpu.sync_copy(ones_hbm, ones_v)
        pltpu.sync_copy(zeros_hbm, z8_v)
        for t in range(rch):
            pltpu.sync_copy(z8_v, acc.at[pl.ds(row0 + t * CH, CH)])
        plsc.subcore_barrier()

        idxb = (idx0, idx1)
        isem = (is0, is1)

        def fetch(j, slot):
            pltpu.async_copy(idx_hbm.at[base + j], idxb[slot], isem[slot])

        def wait_fetch(j, slot):
            pltpu.make_async_copy(idx_hbm.at[base + j], idxb[slot],
                                  isem[slot]).wait()

        def scat(slot):
            pltpu.sync_copy(ones_v, acc.at[idxb[slot].at[1]], add=True)

        fetch(0, 0)
        fetch(1, 1)

        def step(p, carry):
            j = p * 2
            wait_fetch(j, 0)
            scat(0)
            fetch(j + 2, 0)
            wait_fetch(j + 1, 1)
            scat(1)
            fetch(j + 3, 1)
            return carry

        lax.fori_loop(0, nch // 2 - 1, step, 0)
        wait_fetch(nch - 2, 0)
        scat(0)
        wait_fetch(nch - 1, 1)
        scat(1)

        plsc.subcore_barrier()
        for t in range(rch):
            sl = pl.ds(row0 + t * CH, CH)
            pltpu.sync_copy(acc.at[sl], z8_v)
            pltpu.sync_copy(z8_v, out_hbm.at[c, sl])

    f = pl.kernel(
        body,
        out_type=jax.ShapeDtypeStruct((NC, n_pad, 8), _f32),
        mesh=_sc_mesh(),
        scratch_types=[
            pltpu.VMEM((2, CH), jnp.int32),
            pltpu.VMEM((2, CH), jnp.int32),
            pltpu.VMEM((CH, 8), _f32),
            pltpu.VMEM((CH, 8), _f32),
            pltpu.SemaphoreType.DMA,
            pltpu.SemaphoreType.DMA,
            pltpu.VMEM_SHARED((n_pad, 8), _f32),
        ],
        compiler_params=pltpu.CompilerParams(use_tc_tiling_on_sc=False),
    )
    return f(idxc, ones8, zeros8)


def _sc_agg(table, idxc, zrows, n_pad, split0):
    n, d = table.shape
    pairs = idxc.shape[0] // NS
    nch0 = max(2, int(pairs * split0) // 2 * 2)
    nch1 = pairs - nch0
    assert nch1 >= 2
    rps = n_pad // NS
    rch = rps // CH

    def body(table_hbm, idx_hbm, zeros_hbm, out_hbm,
             idx0, idx1, rows0, rows1, sem0, sem1, acc):
        c = lax.axis_index("c")
        s = lax.axis_index("s")
        row0 = s * rps
        pltpu.sync_copy(zeros_hbm, rows0)
        for t in range(rch):
            pltpu.sync_copy(rows0, acc.at[pl.ds(row0 + t * CH, CH)])
        plsc.subcore_barrier()

        rows = (rows0, rows1)
        idxb = (idx0, idx1)
        sems = (sem0, sem1)

        def make_pipeline(nch, base):
            def fetch_start(j, slot):
                pltpu.sync_copy(idx_hbm.at[base + j], idxb[slot])
                pltpu.async_copy(table_hbm.at[idxb[slot].at[0]], rows[slot],
                                 sems[slot])

            def finish(slot):
                pltpu.make_async_copy(table_hbm.at[idxb[slot].at[0]],
                                      rows[slot], sems[slot]).wait()
                pltpu.sync_copy(rows[slot], acc.at[idxb[slot].at[1]], add=True)

            def run():
                fetch_start(0, 0)

                def step(p, carry):
                    j = p * 2
                    fetch_start(j + 1, 1)
                    finish(0)
                    fetch_start(j + 2, 0)
                    finish(1)
                    return carry

                lax.fori_loop(0, nch // 2 - 1, step, 0)
                fetch_start(nch - 1, 1)
                finish(0)
                finish(1)

            return run

        run0 = make_pipeline(nch0, s * nch0)
        run1 = make_pipeline(nch1, NS * nch0 + s * nch1)
        lax.cond(c == 0, run0, run1)

        plsc.subcore_barrier()
        for t in range(rch):
            sl = pl.ds(row0 + t * CH, CH)
            pltpu.sync_copy(acc.at[sl], rows0)
            pltpu.sync_copy(rows0, out_hbm.at[c, sl])

    f = pl.kernel(
        body,
        out_type=jax.ShapeDtypeStruct((NC, n_pad, d), _f32),
        mesh=_sc_mesh(),
        scratch_types=[
            pltpu.VMEM((2, CH), jnp.int32),
            pltpu.VMEM((2, CH), jnp.int32),
            pltpu.VMEM((CH, d), _f32),
            pltpu.VMEM((CH, d), _f32),
            pltpu.SemaphoreType.DMA,
            pltpu.SemaphoreType.DMA,
            pltpu.VMEM_SHARED((n_pad, d), _f32),
        ],
        compiler_params=pltpu.CompilerParams(use_tc_tiling_on_sc=False),
    )
    return f(table, idxc, zrows)


def _block(n):
    for b in (1000, 500, 250, 200, 125, 100, 50, 25, 10, 5):
        if n % b == 0:
            return b
    return n


def _tc_l1(x, w1t, degp):
    n, dx = x.shape
    dd = w1t.shape[1]
    bs = _block(n)

    def body(x_ref, w_ref, d0_ref, d1_ref, hp_ref, dinv_ref):
        deg = d0_ref[0, :, 0:1] + d1_ref[0, :, 0:1] + 1.0
        dinv = lax.rsqrt(deg)
        hp_ref[...] = (
            jnp.dot(x_ref[...], w_ref[...], preferred_element_type=_f32) * dinv
        )
        dinv_ref[...] = jnp.broadcast_to(dinv, dinv_ref.shape)

    return pl.pallas_call(
        body,
        grid=(n // bs,),
        in_specs=[
            pl.BlockSpec((bs, dx), lambda i: (i, 0)),
            pl.BlockSpec((dx, dd), lambda i: (0, 0)),
            pl.BlockSpec((1, bs, 8), lambda i: (0, i, 0)),
            pl.BlockSpec((1, bs, 8), lambda i: (1, i, 0)),
        ],
        out_specs=[
            pl.BlockSpec((bs, dd), lambda i: (i, 0)),
            pl.BlockSpec((bs, 8), lambda i: (i, 0)),
        ],
        out_shape=[
            jax.ShapeDtypeStruct((n, dd), _f32),
            jax.ShapeDtypeStruct((n, 8), _f32),
        ],
    )(x, w1t, degp, degp)


def _tc_combine(parts, hp, dinv8, b):
    n, d = hp.shape
    bs = _block(n)

    def body(p0_ref, p1_ref, hp_ref, dinv_ref, b_ref, z_ref, st_ref):
        i = pl.program_id(0)
        z = (dinv_ref[:, 0:1] * (p0_ref[0] + p1_ref[0] + hp_ref[...])
             + b_ref[...])
        z_ref[...] = z
        part = jnp.concatenate(
            [jnp.sum(z, 0, keepdims=True), jnp.sum(z * z, 0, keepdims=True)], 0
        )

        @pl.when(i == 0)
        def _init():
            st_ref[...] = part

        @pl.when(i > 0)
        def _acc():
            st_ref[...] += part

    return pl.pallas_call(
        body,
        grid=(n // bs,),
        in_specs=[
            pl.BlockSpec((1, bs, d), lambda i: (0, i, 0)),
            pl.BlockSpec((1, bs, d), lambda i: (1, i, 0)),
            pl.BlockSpec((bs, d), lambda i: (i, 0)),
            pl.BlockSpec((bs, 8), lambda i: (i, 0)),
            pl.BlockSpec((1, d), lambda i: (0, 0)),
        ],
        out_specs=[
            pl.BlockSpec((bs, d), lambda i: (i, 0)),
            pl.BlockSpec((2, d), lambda i: (0, 0)),
        ],
        out_shape=[
            jax.ShapeDtypeStruct((n, d), _f32),
            jax.ShapeDtypeStruct((2, d), _f32),
        ],
    )(parts, parts, hp, dinv8, b)


def _tc_bnmm(z, st, g, beta, wt, dinv8=None, bias=None):
    n, d = z.shape
    do = wt.shape[1]
    bs = _block(n)
    nf = float(n)

    def body(*refs):
        if dinv8 is not None:
            z_ref, st_ref, g_ref, beta_ref, w_ref, dv_ref, o_ref = refs
        elif bias is not None:
            z_ref, st_ref, g_ref, beta_ref, w_ref, b_ref, o_ref = refs
        else:
            z_ref, st_ref, g_ref, beta_ref, w_ref, o_ref = refs
        mean = st_ref[0:1, :] / nf
        var = st_ref[1:2, :] / nf - mean * mean
        sc = g_ref[...] * lax.rsqrt(var + 1e-5)
        h = jnp.maximum(z_ref[...] * sc + (beta_ref[...] - mean * sc), 0.0)
        o = jnp.dot(h, w_ref[...], preferred_element_type=_f32)
        if dinv8 is not None:
            o = o * dv_ref[:, 0:1]
        if bias is not None:
            o = o + b_ref[...]
        o_ref[...] = o

    in_specs = [
        pl.BlockSpec((bs, d), lambda i: (i, 0)),
        pl.BlockSpec((2, d), lambda i: (0, 0)),
        pl.BlockSpec((1, d), lambda i: (0, 0)),
        pl.BlockSpec((1, d), lambda i: (0, 0)),
        pl.BlockSpec((d, do), lambda i: (0, 0)),
    ]
    args = [z, st, g, beta, wt]
    if dinv8 is not None:
        in_specs.append(pl.BlockSpec((bs, 8), lambda i: (i, 0)))
        args.append(dinv8)
    if bias is not None:
        in_specs.append(pl.BlockSpec((1, do), lambda i: (0, 0)))
        args.append(bias)

    return pl.pallas_call(
        body,
        grid=(n // bs,),
        in_specs=in_specs,
        out_specs=pl.BlockSpec((bs, do), lambda i: (i, 0)),
        out_shape=jax.ShapeDtypeStruct((n, do), _f32),
    )(*args)


def _ceil_to(a, m):
    return (a + m - 1) // m * m


def kernel(x, edge_index, W1, b1, g1, beta1, W2, b2, g2, beta2, Wfc, bfc):
    n = x.shape[0]
    e = edge_index.shape[1]
    n_pad = _ceil_to(n + 1, NS * CH)
    e_pad = _ceil_to(e, NW * CH * 2)

    src_p = jnp.concatenate(
        [edge_index[0], jnp.zeros((e_pad - e,), edge_index.dtype)]
    )
    dst_p = jnp.concatenate(
        [edge_index[1], jnp.full((e_pad - e,), n, edge_index.dtype)]
    )

    idxc = jnp.stack([src_p.reshape(-1, CH), dst_p.reshape(-1, CH)], axis=1)

    ones8 = jnp.ones((CH, 8), _f32)
    zeros8 = jnp.zeros((CH, 8), _f32)
    degp = _sc_deg(idxc, ones8, zeros8, n_pad)

    hp1, dinv8 = _tc_l1(x, W1.T, degp)

    parts1 = _sc_agg(hp1, idxc, jnp.zeros((CH, hp1.shape[1]), _f32), n_pad, 0.87)
    z1, st1 = _tc_combine(parts1, hp1, dinv8, b1.reshape(1, -1))
    hp2 = _tc_bnmm(z1, st1, g1.reshape(1, -1), beta1.reshape(1, -1), W2.T, dinv8=dinv8)

    parts2 = _sc_agg(hp2, idxc, jnp.zeros((CH, hp2.shape[1]), _f32), n_pad, 0.74)
    z2, st2 = _tc_combine(parts2, hp2, dinv8, b2.reshape(1, -1))
    return _tc_bnmm(z2, st2, g2.reshape(1, -1), beta2.reshape(1, -1), Wfc.T, bias=bfc.reshape(1, -1))
